# baseline (speedup 1.0000x reference)
import functools
import math

import jax
import jax.numpy as jnp
import numpy as np
from jax import lax
from jax.experimental import pallas as pl
from jax.experimental.pallas import tpu as pltpu
from jax.experimental.pallas import tpu_sc as plsc

B, T, D = 1, 2048, 1024
H, DH, DC = 16, 64, 256
HID, NG, EPG, NE, TOPK = 2048, 2, 4, 8, 2
BT = 256
NBT = T // BT

_f32 = jnp.float32


def _dotT(a, b):
    return jax.lax.dot_general(a, b, (((1,), (1,)), ((), ())),
                               preferred_element_type=_f32)


def _rotary_tables():
    inv_freq = 1.0 / (10000.0 ** (np.arange(0, DH, 2, dtype=np.float64) / DH))
    pos = np.arange(T, dtype=np.float64)
    ang = np.einsum('i,j->ij', pos, inv_freq)
    cos = np.cos(ang).astype(np.float32)
    sin = np.sin(ang).astype(np.float32)
    return jnp.asarray(cos), jnp.asarray(sin)


def _head_perm():
    p = np.concatenate([np.arange(0, DH, 2), np.arange(1, DH, 2)])
    full = np.concatenate([h * DH + p for h in range(H)])
    return jnp.asarray(full, dtype=jnp.int32)


def _proj_body(x_ref, n1_ref, wq_ref, wkc_ref, wvc_ref, wk_ref, wv_ref,
               cos_ref, sin_ref, q_ref, k_ref, v_ref):
    x = x_ref[...]
    var = jnp.mean(x * x, axis=-1, keepdims=True)
    h = x / jnp.sqrt(var + 1e-6) * n1_ref[...]
    q = _dotT(h, wq_ref[...])
    kc = _dotT(h, wkc_ref[...])
    vc = _dotT(h, wvc_ref[...])
    k = _dotT(kc, wk_ref[...])
    v = _dotT(vc, wv_ref[...])
    cos = cos_ref[...]
    sin = sin_ref[...]

    for hh in range(H):
        for z, ref in ((q, q_ref), (k, k_ref)):
            x1 = z[:, hh * DH:hh * DH + DH // 2]
            x2 = z[:, hh * DH + DH // 2:(hh + 1) * DH]
            ref[hh] = jnp.concatenate(
                [x1 * cos - x2 * sin, x1 * sin + x2 * cos], axis=1)
        v_ref[hh] = v[:, hh * DH:(hh + 1) * DH]


def _proj(x2d, n1, wq_p, wkc, wvc, wk_p, wv, cos, sin):
    grid = (NBT,)
    return pl.pallas_call(
        _proj_body,
        grid=grid,
        in_specs=[
            pl.BlockSpec((BT, D), lambda i: (i, 0)),
            pl.BlockSpec((1, D), lambda i: (0, 0)),
            pl.BlockSpec((D, D), lambda i: (0, 0)),
            pl.BlockSpec((DC, D), lambda i: (0, 0)),
            pl.BlockSpec((DC, D), lambda i: (0, 0)),
            pl.BlockSpec((D, DC), lambda i: (0, 0)),
            pl.BlockSpec((D, DC), lambda i: (0, 0)),
            pl.BlockSpec((BT, DH // 2), lambda i: (i, 0)),
            pl.BlockSpec((BT, DH // 2), lambda i: (i, 0)),
        ],
        out_specs=[
            pl.BlockSpec((H, BT, DH), lambda i: (0, i, 0)),
            pl.BlockSpec((H, BT, DH), lambda i: (0, i, 0)),
            pl.BlockSpec((H, BT, DH), lambda i: (0, i, 0)),
        ],
        out_shape=[jax.ShapeDtypeStruct((H, T, DH), _f32)] * 3,
    )(x2d, n1, wq_p, wkc, wvc, wk_p, wv, cos, sin)


def _attn_body(q_ref, k_ref, v_ref, o_ref):
    q = q_ref[0]
    s = jax.lax.dot_general(q, k_ref[0], (((1,), (1,)), ((), ())),
                            preferred_element_type=_f32)
    s = s * (1.0 / math.sqrt(DH))
    m = jnp.max(s, axis=-1, keepdims=True)
    p = jnp.exp(s - m)
    l = jnp.sum(p, axis=-1, keepdims=True)
    o = jax.lax.dot_general(p, v_ref[0], (((1,), (0,)), ((), ())),
                            preferred_element_type=_f32)
    o_ref[0] = o / l


def _attn(q, k, v):
    grid = (H, NBT)
    return pl.pallas_call(
        _attn_body,
        grid=grid,
        in_specs=[
            pl.BlockSpec((1, BT, DH), lambda h, i: (h, i, 0)),
            pl.BlockSpec((1, T, DH), lambda h, i: (h, 0, 0)),
            pl.BlockSpec((1, T, DH), lambda h, i: (h, 0, 0)),
        ],
        out_specs=pl.BlockSpec((1, BT, DH), lambda h, i: (h, i, 0)),
        out_shape=jax.ShapeDtypeStruct((H, T, DH), _f32),
    )(q, k, v)


def _post_body(o_ref, x_ref, wo_ref, n2_ref, xa_ref, h2_ref):
    o2 = jnp.concatenate([o_ref[hh] for hh in range(H)], axis=1)
    xa = x_ref[...] + _dotT(o2, wo_ref[...])
    xa_ref[...] = xa
    var = jnp.mean(xa * xa, axis=-1, keepdims=True)
    h2_ref[...] = xa / jnp.sqrt(var + 1e-6) * n2_ref[...]


def _post(o, x2d, wo, n2):
    return pl.pallas_call(
        _post_body,
        grid=(NBT,),
        in_specs=[
            pl.BlockSpec((H, BT, DH), lambda i: (0, i, 0)),
            pl.BlockSpec((BT, D), lambda i: (i, 0)),
            pl.BlockSpec((D, D), lambda i: (0, 0)),
            pl.BlockSpec((1, D), lambda i: (0, 0)),
        ],
        out_specs=[
            pl.BlockSpec((BT, D), lambda i: (i, 0)),
            pl.BlockSpec((BT, D), lambda i: (i, 0)),
        ],
        out_shape=[jax.ShapeDtypeStruct((T, D), _f32)] * 2,
    )(o, x2d, wo, n2)


def _gates_body(h2_ref, gg_ref, eg_ref, gb_ref, eb_ref, topi_ref, topv_ref):
    h2 = h2_ref[...]
    glog = _dotT(h2, gg_ref[...]) + gb_ref[...]
    gm = jnp.max(glog, axis=-1, keepdims=True)
    ge = jnp.exp(glog - gm)
    gprobs = ge / jnp.sum(ge, axis=-1, keepdims=True)
    g_idx = (gprobs[:, 1:2] > gprobs[:, 0:1]).astype(jnp.int32)
    g_prob = jnp.max(gprobs, axis=-1, keepdims=True)

    elog = _dotT(h2, eg_ref[...]) + eb_ref[...]
    idx8 = jax.lax.broadcasted_iota(jnp.int32, (BT, NE), 1)
    allowed = (idx8 // EPG) == g_idx
    masked = jnp.where(allowed, elog, -jnp.inf)
    m = jnp.max(masked, axis=-1, keepdims=True)
    ex = jnp.exp(masked - m)
    eprobs = ex / jnp.sum(ex, axis=-1, keepdims=True)
    p = eprobs * g_prob

    m1 = jnp.max(p, axis=-1, keepdims=True)
    i1 = jnp.min(jnp.where(p == m1, idx8, NE), axis=-1, keepdims=True)
    p2 = jnp.where(idx8 == i1, -1.0, p)
    m2 = jnp.max(p2, axis=-1, keepdims=True)
    i2 = jnp.min(jnp.where(p2 == m2, idx8, NE), axis=-1, keepdims=True)
    topi_ref[...] = jnp.concatenate([i1, i2], axis=1)
    topv_ref[...] = jnp.concatenate([m1, m2], axis=1)


def _gates(h2, gg, eg, gb, eb):
    return pl.pallas_call(
        _gates_body,
        grid=(NBT,),
        in_specs=[
            pl.BlockSpec((BT, D), lambda i: (i, 0)),
            pl.BlockSpec((NG, D), lambda i: (0, 0)),
            pl.BlockSpec((NE, D), lambda i: (0, 0)),
            pl.BlockSpec((1, NG), lambda i: (0, 0)),
            pl.BlockSpec((1, NE), lambda i: (0, 0)),
        ],
        out_specs=[
            pl.BlockSpec((BT, TOPK), lambda i: (i, 0)),
            pl.BlockSpec((BT, TOPK), lambda i: (i, 0)),
        ],
        out_shape=[
            jax.ShapeDtypeStruct((T, TOPK), jnp.int32),
            jax.ShapeDtypeStruct((T, TOPK), _f32),
        ],
    )(h2, gg, eg, gb, eb)


def _swiglu(z):
    a = z[:, :HID]
    b = z[:, HID:]
    return a * jax.lax.logistic(a) * b


def _ffn_up_body(h2_ref, w_ref, out_ref):
    out_ref[...] = _swiglu(_dotT(h2_ref[...], w_ref[...]))


def _ffn_up(h2, w):
    return pl.pallas_call(
        _ffn_up_body,
        grid=(NBT,),
        in_specs=[
            pl.BlockSpec((BT, D), lambda i: (i, 0)),
            pl.BlockSpec((2 * HID, D), lambda i: (0, 0)),
        ],
        out_specs=pl.BlockSpec((BT, HID), lambda i: (i, 0)),
        out_shape=jax.ShapeDtypeStruct((T, HID), _f32),
    )(h2, w)


TILE = 256
NTILES = 24
P = NTILES * TILE


def _route_body(topi_ref, d0_ref, d1_ref, te_ref):
    topi = topi_ref[...]
    idx8a = jax.lax.broadcasted_iota(jnp.int32, (T, NE), 1)
    oh0 = (topi[:, 0:1] == idx8a).astype(_f32)
    oh1 = (topi[:, 1:2] == idx8a).astype(_f32)
    oh2 = oh0 + oh1

    r = jax.lax.broadcasted_iota(jnp.int32, (TILE, TILE), 0)
    c = jax.lax.broadcasted_iota(jnp.int32, (TILE, TILE), 1)
    ltri = (r > c).astype(_f32)
    ones_row = jnp.ones((1, TILE), _f32)

    excl_parts = []
    carry = jnp.zeros((1, NE), _f32)
    for b in range(T // TILE):
        blk = oh2[b * TILE:(b + 1) * TILE, :]
        excl_parts.append(
            jax.lax.dot_general(ltri, blk, (((1,), (0,)), ((), ())),
                                preferred_element_type=_f32) + carry)
        carry = carry + jax.lax.dot_general(
            ones_row, blk, (((1,), (0,)), ((), ())),
            preferred_element_type=_f32)
    excl = jnp.concatenate(excl_parts, axis=0)
    totals = carry

    tiles = jnp.floor((totals + (TILE - 1)) * (1.0 / TILE))
    e1 = jax.lax.broadcasted_iota(jnp.int32, (NE, NE), 0)
    e2 = jax.lax.broadcasted_iota(jnp.int32, (NE, NE), 1)
    u8 = (e1 < e2).astype(_f32)
    pofft = jax.lax.dot_general(tiles, u8, (((1,), (0,)), ((), ())),
                                preferred_element_type=_f32)
    poffe = pofft * float(TILE)

    base0 = jnp.sum(oh0 * poffe, axis=1, keepdims=True)
    rank0 = jnp.sum(oh0 * excl, axis=1, keepdims=True)
    base1 = jnp.sum(oh1 * poffe, axis=1, keepdims=True)
    rank1 = jnp.sum(oh1 * excl, axis=1, keepdims=True)
    d0_ref[...] = (base0 + rank0).astype(jnp.int32)
    d1_ref[...] = (base1 + rank1).astype(jnp.int32)

    t32 = jax.lax.broadcasted_iota(jnp.int32, (32, NE), 0).astype(_f32)
    covered = (t32 >= pofft).astype(_f32)
    te_ref[...] = (jnp.sum(covered, axis=1, keepdims=True) - 1.0
                   ).astype(jnp.int32)


def _route(topi):
    return pl.pallas_call(
        _route_body,
        grid=(1,),
        in_specs=[pl.BlockSpec((T, TOPK), lambda i: (0, 0))],
        out_specs=[
            pl.BlockSpec((T, 1), lambda i: (0, 0)),
            pl.BlockSpec((T, 1), lambda i: (0, 0)),
            pl.BlockSpec((32, 1), lambda i: (0, 0)),
        ],
        out_shape=[
            jax.ShapeDtypeStruct((T, 1), jnp.int32),
            jax.ShapeDtypeStruct((T, 1), jnp.int32),
            jax.ShapeDtypeStruct((32, 1), jnp.int32),
        ],
    )(topi)


_SC_MESH = dict(core_axis_name="c", subcore_axis_name="s")
_NW = 32


def _sc_scatter_hmid(h_mid, d0, d1):
    rows_per_w = T // _NW
    nch = rows_per_w // 16

    @functools.partial(
        pl.kernel,
        mesh=plsc.VectorSubcoreMesh(**_SC_MESH),
        out_type=jax.ShapeDtypeStruct((P, HID), _f32),
        scratch_types=[
            pltpu.VMEM((16,), jnp.int32),
            pltpu.VMEM((16, HID), _f32),
            pltpu.SemaphoreType.DMA,
        ],
    )
    def k(hm_hbm, d0_hbm, d1_hbm, out_hbm, idx_v, rows_v, sem):
        wid = lax.axis_index("s") * 2 + lax.axis_index("c")
        for d_hbm in (d0_hbm, d1_hbm):
            for ch in range(nch):
                s0 = wid * rows_per_w + ch * 16
                pltpu.sync_copy(d_hbm.at[pl.ds(s0, 16)], idx_v)
                idx = idx_v[...]
                pltpu.sync_copy(hm_hbm.at[pl.ds(s0, 16)], rows_v)
                pltpu.async_copy(rows_v, out_hbm.at[idx], sem).wait()

    return k(h_mid, d0, d1)


def _sc_gather_z(y, d0, d1):
    rows_per_w = T // _NW
    nch = rows_per_w // 16

    @functools.partial(
        pl.kernel,
        mesh=plsc.VectorSubcoreMesh(**_SC_MESH),
        out_type=jax.ShapeDtypeStruct((TOPK * T, D), _f32),
        scratch_types=[
            pltpu.VMEM((16,), jnp.int32),
            pltpu.VMEM((16, D), _f32),
            pltpu.SemaphoreType.DMA,
        ],
    )
    def k(y_hbm, d0_hbm, d1_hbm, z_hbm, idx_v, rows_v, sem):
        wid = lax.axis_index("s") * 2 + lax.axis_index("c")
        for kk, d_hbm in ((0, d0_hbm), (1, d1_hbm)):
            for ch in range(nch):
                s0 = wid * rows_per_w + ch * 16
                pltpu.sync_copy(d_hbm.at[pl.ds(s0, 16)], idx_v)
                idx = idx_v[...]
                pltpu.async_copy(y_hbm.at[idx], rows_v, sem).wait()
                pltpu.sync_copy(rows_v, z_hbm.at[pl.ds(kk * T + s0, 16)])

    return k(y, d0, d1)


def _gmm_body(te_ref, hg_ref, w2_ref, y_ref):
    del te_ref
    y_ref[...] = jax.lax.dot_general(
        hg_ref[...], w2_ref[0], (((1,), (1,)), ((), ())),
        preferred_element_type=_f32)


def _gmm(hg, w2, te):
    grid_spec = pltpu.PrefetchScalarGridSpec(
        num_scalar_prefetch=1,
        grid=(NTILES,),
        in_specs=[
            pl.BlockSpec((TILE, HID), lambda t, te_ref: (t, 0)),
            pl.BlockSpec((1, D, HID), lambda t, te_ref: (te_ref[t], 0, 0)),
        ],
        out_specs=pl.BlockSpec((TILE, D), lambda t, te_ref: (t, 0)),
    )
    return pl.pallas_call(
        _gmm_body,
        grid_spec=grid_spec,
        out_shape=jax.ShapeDtypeStruct((P, D), _f32),
    )(te, hg, w2)


def _final_body(xa_ref, hid1_ref, so_ref, z0_ref, z1_ref, tv_ref, out_ref):
    sh = _dotT(hid1_ref[...], so_ref[...])
    tv = tv_ref[...]
    routed = z0_ref[...] * tv[:, 0:1] + z1_ref[...] * tv[:, 1:2]
    out_ref[...] = xa_ref[...] + sh + routed


def _final(xa, hid1, so, z, topv):
    return pl.pallas_call(
        _final_body,
        grid=(NBT,),
        in_specs=[
            pl.BlockSpec((BT, D), lambda i: (i, 0)),
            pl.BlockSpec((BT, HID), lambda i: (i, 0)),
            pl.BlockSpec((D, HID), lambda i: (0, 0)),
            pl.BlockSpec((BT, D), lambda i: (i, 0)),
            pl.BlockSpec((BT, D), lambda i: (i + NBT, 0)),
            pl.BlockSpec((BT, TOPK), lambda i: (i, 0)),
        ],
        out_specs=pl.BlockSpec((BT, D), lambda i: (i, 0)),
        out_shape=jax.ShapeDtypeStruct((T, D), _f32),
    )(xa, hid1, so, z, z, topv)


def kernel(x, Wq, Wk_c, Wv_c, Wk, Wv, Wo, norm1_w, norm2_w, shared_in,
           shared_out, w1_shared, w2_expert, group_gate, expert_gate,
           group_bias, expert_bias):
    x2d = x.reshape(T, D)
    perm = _head_perm()
    wq_p = Wq[perm, :]
    wk_p = Wk[perm, :]
    cos, sin = _rotary_tables()

    q, k, v = _proj(x2d, norm1_w.reshape(1, D), wq_p, Wk_c, Wv_c, wk_p, Wv,
                    cos, sin)
    o = _attn(q, k, v)
    xa, h2 = _post(o, x2d, Wo, norm2_w.reshape(1, D))

    topi, topv = _gates(h2, group_gate, expert_gate, group_bias.reshape(1, NG),
                        expert_bias.reshape(1, NE))
    d0, d1, te32 = _route(topi)
    h_mid = _ffn_up(h2, w1_shared)
    d0f = d0.reshape(T)
    d1f = d1.reshape(T)
    hg = _sc_scatter_hmid(h_mid, d0f, d1f)
    hid1 = _ffn_up(h2, shared_in)
    te = te32.reshape(32)[:NTILES]
    y = _gmm(hg, w2_expert, te)
    z = _sc_gather_z(y, d0f, d1f)
    out = _final(xa, hid1, shared_out, z, topv)
    return out.reshape(B, T, D)


# speedup vs baseline: 1.0862x; 1.0862x over previous
import functools
import math

import jax
import jax.numpy as jnp
import numpy as np
from jax.experimental import pallas as pl
from jax.experimental.pallas import tpu as pltpu

B, T, D = 1, 2048, 1024
H, DH, DC = 16, 64, 256
HID, NG, EPG, NE, TOPK = 2048, 2, 4, 8, 2
BT = 256
NBT = T // BT

_f32 = jnp.float32


def _dotT(a, b):
    return jax.lax.dot_general(a, b, (((1,), (1,)), ((), ())),
                               preferred_element_type=_f32)


def _rotary_tables():
    inv_freq = 1.0 / (10000.0 ** (np.arange(0, DH, 2, dtype=np.float64) / DH))
    pos = np.arange(T, dtype=np.float64)
    ang = np.einsum('i,j->ij', pos, inv_freq)
    cos = np.cos(ang).astype(np.float32)
    sin = np.sin(ang).astype(np.float32)
    return jnp.asarray(cos), jnp.asarray(sin)


def _head_perm():
    p = np.concatenate([np.arange(0, DH, 2), np.arange(1, DH, 2)])
    full = np.concatenate([h * DH + p for h in range(H)])
    return jnp.asarray(full, dtype=jnp.int32)


def _proj_body(x_ref, n1_ref, wq_ref, wkc_ref, wvc_ref, wk_ref, wv_ref,
               cos_ref, sin_ref, q_ref, k_ref, v_ref):
    x = x_ref[...]
    var = jnp.mean(x * x, axis=-1, keepdims=True)
    h = x / jnp.sqrt(var + 1e-6) * n1_ref[...]
    q = _dotT(h, wq_ref[...])
    kc = _dotT(h, wkc_ref[...])
    vc = _dotT(h, wvc_ref[...])
    k = _dotT(kc, wk_ref[...])
    v = _dotT(vc, wv_ref[...])
    cos = cos_ref[...]
    sin = sin_ref[...]

    for hh in range(H):
        for z, ref in ((q, q_ref), (k, k_ref)):
            x1 = z[:, hh * DH:hh * DH + DH // 2]
            x2 = z[:, hh * DH + DH // 2:(hh + 1) * DH]
            ref[hh] = jnp.concatenate(
                [x1 * cos - x2 * sin, x1 * sin + x2 * cos], axis=1)
        v_ref[hh] = v[:, hh * DH:(hh + 1) * DH]


def _proj(x2d, n1, wq_p, wkc, wvc, wk_p, wv, cos, sin):
    grid = (NBT,)
    return pl.pallas_call(
        _proj_body,
        grid=grid,
        in_specs=[
            pl.BlockSpec((BT, D), lambda i: (i, 0)),
            pl.BlockSpec((1, D), lambda i: (0, 0)),
            pl.BlockSpec((D, D), lambda i: (0, 0)),
            pl.BlockSpec((DC, D), lambda i: (0, 0)),
            pl.BlockSpec((DC, D), lambda i: (0, 0)),
            pl.BlockSpec((D, DC), lambda i: (0, 0)),
            pl.BlockSpec((D, DC), lambda i: (0, 0)),
            pl.BlockSpec((BT, DH // 2), lambda i: (i, 0)),
            pl.BlockSpec((BT, DH // 2), lambda i: (i, 0)),
        ],
        out_specs=[
            pl.BlockSpec((H, BT, DH), lambda i: (0, i, 0)),
            pl.BlockSpec((H, BT, DH), lambda i: (0, i, 0)),
            pl.BlockSpec((H, BT, DH), lambda i: (0, i, 0)),
        ],
        out_shape=[jax.ShapeDtypeStruct((H, T, DH), _f32)] * 3,
    )(x2d, n1, wq_p, wkc, wvc, wk_p, wv, cos, sin)


def _attn_body(q_ref, k_ref, v_ref, o_ref):
    q = q_ref[0]
    s = jax.lax.dot_general(q, k_ref[0], (((1,), (1,)), ((), ())),
                            preferred_element_type=_f32)
    s = s * (1.0 / math.sqrt(DH))
    m = jnp.max(s, axis=-1, keepdims=True)
    p = jnp.exp(s - m)
    l = jnp.sum(p, axis=-1, keepdims=True)
    o = jax.lax.dot_general(p, v_ref[0], (((1,), (0,)), ((), ())),
                            preferred_element_type=_f32)
    o_ref[0] = o / l


def _attn(q, k, v):
    grid = (H, NBT)
    return pl.pallas_call(
        _attn_body,
        grid=grid,
        in_specs=[
            pl.BlockSpec((1, BT, DH), lambda h, i: (h, i, 0)),
            pl.BlockSpec((1, T, DH), lambda h, i: (h, 0, 0)),
            pl.BlockSpec((1, T, DH), lambda h, i: (h, 0, 0)),
        ],
        out_specs=pl.BlockSpec((1, BT, DH), lambda h, i: (h, i, 0)),
        out_shape=jax.ShapeDtypeStruct((H, T, DH), _f32),
    )(q, k, v)


def _post_body(o_ref, x_ref, wo_ref, n2_ref, gg_ref, eg_ref, gb_ref, eb_ref,
               xa_ref, h2_ref, wgt_ref):
    o2 = jnp.concatenate([o_ref[hh] for hh in range(H)], axis=1)
    xa = x_ref[...] + _dotT(o2, wo_ref[...])
    xa_ref[...] = xa
    var = jnp.mean(xa * xa, axis=-1, keepdims=True)
    h2 = xa / jnp.sqrt(var + 1e-6) * n2_ref[...]
    h2_ref[...] = h2

    glog = _dotT(h2, gg_ref[...]) + gb_ref[...]
    gm = jnp.max(glog, axis=-1, keepdims=True)
    ge = jnp.exp(glog - gm)
    gprobs = ge / jnp.sum(ge, axis=-1, keepdims=True)
    g_idx = (gprobs[:, 1:2] > gprobs[:, 0:1]).astype(jnp.int32)
    g_prob = jnp.max(gprobs, axis=-1, keepdims=True)

    elog = _dotT(h2, eg_ref[...]) + eb_ref[...]
    idx8 = jax.lax.broadcasted_iota(jnp.int32, (BT, NE), 1)
    allowed = (idx8 // EPG) == g_idx
    masked = jnp.where(allowed, elog, -jnp.inf)
    m = jnp.max(masked, axis=-1, keepdims=True)
    ex = jnp.exp(masked - m)
    eprobs = ex / jnp.sum(ex, axis=-1, keepdims=True)
    p = eprobs * g_prob

    m1 = jnp.max(p, axis=-1, keepdims=True)
    i1 = jnp.min(jnp.where(p == m1, idx8, NE), axis=-1, keepdims=True)
    p2 = jnp.where(idx8 == i1, -1.0, p)
    m2 = jnp.max(p2, axis=-1, keepdims=True)
    i2 = jnp.min(jnp.where(p2 == m2, idx8, NE), axis=-1, keepdims=True)
    wgt = jnp.where(idx8 == i1, m1, 0.0) + jnp.where(idx8 == i2, m2, 0.0)
    wgt_ref[...] = wgt


def _post(o, x2d, wo, n2, gg, eg, gb, eb):
    return pl.pallas_call(
        _post_body,
        grid=(NBT,),
        in_specs=[
            pl.BlockSpec((H, BT, DH), lambda i: (0, i, 0)),
            pl.BlockSpec((BT, D), lambda i: (i, 0)),
            pl.BlockSpec((D, D), lambda i: (0, 0)),
            pl.BlockSpec((1, D), lambda i: (0, 0)),
            pl.BlockSpec((NG, D), lambda i: (0, 0)),
            pl.BlockSpec((NE, D), lambda i: (0, 0)),
            pl.BlockSpec((1, NG), lambda i: (0, 0)),
            pl.BlockSpec((1, NE), lambda i: (0, 0)),
        ],
        out_specs=[
            pl.BlockSpec((BT, D), lambda i: (i, 0)),
            pl.BlockSpec((BT, D), lambda i: (i, 0)),
            pl.BlockSpec((BT, NE), lambda i: (i, 0)),
        ],
        out_shape=[
            jax.ShapeDtypeStruct((T, D), _f32),
            jax.ShapeDtypeStruct((T, D), _f32),
            jax.ShapeDtypeStruct((T, NE), _f32),
        ],
    )(o, x2d, wo, n2, gg, eg, gb, eb)


def _swiglu(z):
    a = z[:, :HID]
    b = z[:, HID:]
    return a * jax.lax.logistic(a) * b


def _ffn_up_body(h2_ref, w_ref, out_ref):
    out_ref[...] = _swiglu(_dotT(h2_ref[...], w_ref[...]))


def _ffn_up(h2, w):
    return pl.pallas_call(
        _ffn_up_body,
        grid=(NBT,),
        in_specs=[
            pl.BlockSpec((BT, D), lambda i: (i, 0)),
            pl.BlockSpec((2 * HID, D), lambda i: (0, 0)),
        ],
        out_specs=pl.BlockSpec((BT, HID), lambda i: (i, 0)),
        out_shape=jax.ShapeDtypeStruct((T, HID), _f32),
    )(h2, w)


def _ffn_shared_body(h2_ref, w_ref, so_ref, sh_ref):
    hid = _swiglu(_dotT(h2_ref[...], w_ref[...]))
    sh_ref[...] = _dotT(hid, so_ref[...])


def _ffn_shared(h2, w, so):
    return pl.pallas_call(
        _ffn_shared_body,
        grid=(NBT,),
        in_specs=[
            pl.BlockSpec((BT, D), lambda i: (i, 0)),
            pl.BlockSpec((2 * HID, D), lambda i: (0, 0)),
            pl.BlockSpec((D, HID), lambda i: (0, 0)),
        ],
        out_specs=pl.BlockSpec((BT, D), lambda i: (i, 0)),
        out_shape=jax.ShapeDtypeStruct((T, D), _f32),
    )(h2, w, so)


def _routed_body(hm_ref, w2_ref, wgt_ref, out_ref, acc_ref):
    e = pl.program_id(0)
    idx8 = jax.lax.broadcasted_iota(jnp.int32, (T, NE), 1)
    wcol = jnp.sum(jnp.where(idx8 == e, wgt_ref[...], 0.0), axis=1,
                   keepdims=True)
    y = jax.lax.dot_general(hm_ref[...], w2_ref[0], (((1,), (1,)), ((), ())),
                            preferred_element_type=_f32) * wcol

    @pl.when(e == 0)
    def _():
        acc_ref[...] = y

    @pl.when(e > 0)
    def _():
        acc_ref[...] += y

    @pl.when(e == NE - 1)
    def _():
        out_ref[...] = acc_ref[...]


def _routed(h_mid, w2, wgt):
    return pl.pallas_call(
        _routed_body,
        grid=(NE,),
        in_specs=[
            pl.BlockSpec((T, HID), lambda e: (0, 0)),
            pl.BlockSpec((1, D, HID), lambda e: (e, 0, 0)),
            pl.BlockSpec((T, NE), lambda e: (0, 0)),
        ],
        out_specs=pl.BlockSpec((T, D), lambda e: (0, 0)),
        out_shape=jax.ShapeDtypeStruct((T, D), _f32),
        scratch_shapes=[pltpu.VMEM((T, D), _f32)],
    )(h_mid, w2, wgt)


def _final_body(xa_ref, sh_ref, routed_ref, out_ref):
    out_ref[...] = xa_ref[...] + sh_ref[...] + routed_ref[...]


def _final(xa, sh, routed):
    return pl.pallas_call(
        _final_body,
        grid=(NBT,),
        in_specs=[
            pl.BlockSpec((BT, D), lambda i: (i, 0)),
            pl.BlockSpec((BT, D), lambda i: (i, 0)),
            pl.BlockSpec((BT, D), lambda i: (i, 0)),
        ],
        out_specs=pl.BlockSpec((BT, D), lambda i: (i, 0)),
        out_shape=jax.ShapeDtypeStruct((T, D), _f32),
    )(xa, sh, routed)


def kernel(x, Wq, Wk_c, Wv_c, Wk, Wv, Wo, norm1_w, norm2_w, shared_in,
           shared_out, w1_shared, w2_expert, group_gate, expert_gate,
           group_bias, expert_bias):
    x2d = x.reshape(T, D)
    perm = _head_perm()
    wq_p = Wq[perm, :]
    wk_p = Wk[perm, :]
    cos, sin = _rotary_tables()

    q, k, v = _proj(x2d, norm1_w.reshape(1, D), wq_p, Wk_c, Wv_c, wk_p, Wv,
                    cos, sin)
    o = _attn(q, k, v)
    xa, h2, wgt = _post(o, x2d, Wo, norm2_w.reshape(1, D), group_gate,
                        expert_gate, group_bias.reshape(1, NG),
                        expert_bias.reshape(1, NE))
    sh = _ffn_shared(h2, shared_in, shared_out)
    h_mid = _ffn_up(h2, w1_shared)
    routed = _routed(h_mid, w2_expert, wgt)
    out = _final(xa, sh, routed)
    return out.reshape(B, T, D)


# speedup vs baseline: 1.0917x; 1.0051x over previous
import functools
import math

import jax
import jax.numpy as jnp
import numpy as np
from jax.experimental import pallas as pl
from jax.experimental.pallas import tpu as pltpu

B, T, D = 1, 2048, 1024
H, DH, DC = 16, 64, 256
HID, NG, EPG, NE, TOPK = 2048, 2, 4, 8, 2
BT = 256
NBT = T // BT

_f32 = jnp.float32


def _dotT(a, b):
    return jax.lax.dot_general(a, b, (((1,), (1,)), ((), ())),
                               preferred_element_type=_f32)


def _rotary_tables():
    inv_freq = 1.0 / (10000.0 ** (np.arange(0, DH, 2, dtype=np.float64) / DH))
    pos = np.arange(T, dtype=np.float64)
    ang = np.einsum('i,j->ij', pos, inv_freq)
    cos = np.cos(ang).astype(np.float32)
    sin = np.sin(ang).astype(np.float32)
    return jnp.asarray(cos), jnp.asarray(sin)


def _head_perm():
    p = np.concatenate([np.arange(0, DH, 2), np.arange(1, DH, 2)])
    full = np.concatenate([h * DH + p for h in range(H)])
    return jnp.asarray(full, dtype=jnp.int32)


def _proj_body(x_ref, n1_ref, wq_ref, wkc_ref, wvc_ref, wk_ref, wv_ref,
               cos_ref, sin_ref, q_ref, k_ref, v_ref):
    x = x_ref[...]
    var = jnp.mean(x * x, axis=-1, keepdims=True)
    h = x / jnp.sqrt(var + 1e-6) * n1_ref[...]
    q = _dotT(h, wq_ref[...])
    kc = _dotT(h, wkc_ref[...])
    vc = _dotT(h, wvc_ref[...])
    k = _dotT(kc, wk_ref[...])
    v = _dotT(vc, wv_ref[...])
    cos = cos_ref[...]
    sin = sin_ref[...]

    for hh in range(H):
        for z, ref in ((q, q_ref), (k, k_ref)):
            x1 = z[:, hh * DH:hh * DH + DH // 2]
            x2 = z[:, hh * DH + DH // 2:(hh + 1) * DH]
            ref[hh] = jnp.concatenate(
                [x1 * cos - x2 * sin, x1 * sin + x2 * cos], axis=1)
        v_ref[hh] = v[:, hh * DH:(hh + 1) * DH]


def _proj(x2d, n1, wq_p, wkc, wvc, wk_p, wv, cos, sin):
    grid = (NBT,)
    return pl.pallas_call(
        _proj_body,
        grid=grid,
        in_specs=[
            pl.BlockSpec((BT, D), lambda i: (i, 0)),
            pl.BlockSpec((1, D), lambda i: (0, 0)),
            pl.BlockSpec((D, D), lambda i: (0, 0)),
            pl.BlockSpec((DC, D), lambda i: (0, 0)),
            pl.BlockSpec((DC, D), lambda i: (0, 0)),
            pl.BlockSpec((D, DC), lambda i: (0, 0)),
            pl.BlockSpec((D, DC), lambda i: (0, 0)),
            pl.BlockSpec((BT, DH // 2), lambda i: (i, 0)),
            pl.BlockSpec((BT, DH // 2), lambda i: (i, 0)),
        ],
        out_specs=[
            pl.BlockSpec((H, BT, DH), lambda i: (0, i, 0)),
            pl.BlockSpec((H, BT, DH), lambda i: (0, i, 0)),
            pl.BlockSpec((H, BT, DH), lambda i: (0, i, 0)),
        ],
        out_shape=[jax.ShapeDtypeStruct((H, T, DH), _f32)] * 3,
    )(x2d, n1, wq_p, wkc, wvc, wk_p, wv, cos, sin)


def _attn_body(q_ref, k_ref, v_ref, o_ref):
    q = q_ref[0]
    s = jax.lax.dot_general(q, k_ref[0], (((1,), (1,)), ((), ())),
                            preferred_element_type=_f32)
    s = s * (1.0 / math.sqrt(DH))
    m = jnp.max(s, axis=-1, keepdims=True)
    p = jnp.exp(s - m)
    l = jnp.sum(p, axis=-1, keepdims=True)
    o = jax.lax.dot_general(p, v_ref[0], (((1,), (0,)), ((), ())),
                            preferred_element_type=_f32)
    o_ref[0] = o / l


def _attn(q, k, v):
    grid = (H, NBT)
    return pl.pallas_call(
        _attn_body,
        grid=grid,
        in_specs=[
            pl.BlockSpec((1, BT, DH), lambda h, i: (h, i, 0)),
            pl.BlockSpec((1, T, DH), lambda h, i: (h, 0, 0)),
            pl.BlockSpec((1, T, DH), lambda h, i: (h, 0, 0)),
        ],
        out_specs=pl.BlockSpec((1, BT, DH), lambda h, i: (h, i, 0)),
        out_shape=jax.ShapeDtypeStruct((H, T, DH), _f32),
    )(q, k, v)


def _post_body(o_ref, x_ref, wo_ref, n2_ref, gg_ref, eg_ref, gb_ref, eb_ref,
               xa_ref, h2_ref, wgt_ref):
    o2 = jnp.concatenate([o_ref[hh] for hh in range(H)], axis=1)
    xa = x_ref[...] + _dotT(o2, wo_ref[...])
    xa_ref[...] = xa
    var = jnp.mean(xa * xa, axis=-1, keepdims=True)
    h2 = xa / jnp.sqrt(var + 1e-6) * n2_ref[...]
    h2_ref[...] = h2

    glog = _dotT(h2, gg_ref[...]) + gb_ref[...]
    gm = jnp.max(glog, axis=-1, keepdims=True)
    ge = jnp.exp(glog - gm)
    gprobs = ge / jnp.sum(ge, axis=-1, keepdims=True)
    g_idx = (gprobs[:, 1:2] > gprobs[:, 0:1]).astype(jnp.int32)
    g_prob = jnp.max(gprobs, axis=-1, keepdims=True)

    elog = _dotT(h2, eg_ref[...]) + eb_ref[...]
    idx8 = jax.lax.broadcasted_iota(jnp.int32, (BT, NE), 1)
    allowed = (idx8 // EPG) == g_idx
    masked = jnp.where(allowed, elog, -jnp.inf)
    m = jnp.max(masked, axis=-1, keepdims=True)
    ex = jnp.exp(masked - m)
    eprobs = ex / jnp.sum(ex, axis=-1, keepdims=True)
    p = eprobs * g_prob

    m1 = jnp.max(p, axis=-1, keepdims=True)
    i1 = jnp.min(jnp.where(p == m1, idx8, NE), axis=-1, keepdims=True)
    p2 = jnp.where(idx8 == i1, -1.0, p)
    m2 = jnp.max(p2, axis=-1, keepdims=True)
    i2 = jnp.min(jnp.where(p2 == m2, idx8, NE), axis=-1, keepdims=True)
    wgt = jnp.where(idx8 == i1, m1, 0.0) + jnp.where(idx8 == i2, m2, 0.0)
    wgt_ref[...] = wgt


def _post(o, x2d, wo, n2, gg, eg, gb, eb):
    return pl.pallas_call(
        _post_body,
        grid=(NBT,),
        in_specs=[
            pl.BlockSpec((H, BT, DH), lambda i: (0, i, 0)),
            pl.BlockSpec((BT, D), lambda i: (i, 0)),
            pl.BlockSpec((D, D), lambda i: (0, 0)),
            pl.BlockSpec((1, D), lambda i: (0, 0)),
            pl.BlockSpec((NG, D), lambda i: (0, 0)),
            pl.BlockSpec((NE, D), lambda i: (0, 0)),
            pl.BlockSpec((1, NG), lambda i: (0, 0)),
            pl.BlockSpec((1, NE), lambda i: (0, 0)),
        ],
        out_specs=[
            pl.BlockSpec((BT, D), lambda i: (i, 0)),
            pl.BlockSpec((BT, D), lambda i: (i, 0)),
            pl.BlockSpec((BT, NE), lambda i: (i, 0)),
        ],
        out_shape=[
            jax.ShapeDtypeStruct((T, D), _f32),
            jax.ShapeDtypeStruct((T, D), _f32),
            jax.ShapeDtypeStruct((T, NE), _f32),
        ],
    )(o, x2d, wo, n2, gg, eg, gb, eb)


def _swiglu(z):
    a = z[:, :HID]
    b = z[:, HID:]
    return a * jax.lax.logistic(a) * b


def _ffn_up_body(h2_ref, w_ref, out_ref):
    out_ref[...] = _swiglu(_dotT(h2_ref[...], w_ref[...]))


def _ffn_up(h2, w):
    return pl.pallas_call(
        _ffn_up_body,
        grid=(NBT,),
        in_specs=[
            pl.BlockSpec((BT, D), lambda i: (i, 0)),
            pl.BlockSpec((2 * HID, D), lambda i: (0, 0)),
        ],
        out_specs=pl.BlockSpec((BT, HID), lambda i: (i, 0)),
        out_shape=jax.ShapeDtypeStruct((T, HID), _f32),
    )(h2, w)


def _ffn_shared_body(h2_ref, w_ref, so_ref, sh_ref):
    hid = _swiglu(_dotT(h2_ref[...], w_ref[...]))
    sh_ref[...] = _dotT(hid, so_ref[...])


def _ffn_shared(h2, w, so):
    return pl.pallas_call(
        _ffn_shared_body,
        grid=(NBT,),
        in_specs=[
            pl.BlockSpec((BT, D), lambda i: (i, 0)),
            pl.BlockSpec((2 * HID, D), lambda i: (0, 0)),
            pl.BlockSpec((D, HID), lambda i: (0, 0)),
        ],
        out_specs=pl.BlockSpec((BT, D), lambda i: (i, 0)),
        out_shape=jax.ShapeDtypeStruct((T, D), _f32),
    )(h2, w, so)


HH = HID // 4
NW1 = 8


RB = 512


def _routed_body(h2_ref, w1q_ref, w2_ref, wgt_ref, out_ref, hm_ref):
    j = pl.program_id(0)

    @pl.when(j < NW1 // 2)
    def _():
        sl = pl.ds(j * HH, HH)
        for r in range(T // RB):
            rs = pl.ds(r * RB, RB)
            hm_ref[rs, sl] = _dotT(h2_ref[rs, :], w1q_ref[...])

    @pl.when((j >= NW1 // 2) & (j < NW1))
    def _():
        sl = pl.ds((j - NW1 // 2) * HH, HH)
        for r in range(T // RB):
            rs = pl.ds(r * RB, RB)
            a = hm_ref[rs, sl]
            hm_ref[rs, sl] = a * jax.lax.logistic(a) * _dotT(h2_ref[rs, :],
                                                             w1q_ref[...])

    @pl.when(j >= NW1)
    def _():
        e = j - NW1
        idx8 = jax.lax.broadcasted_iota(jnp.int32, (T, NE), 1)
        wcol = jnp.sum(jnp.where(idx8 == e, wgt_ref[...], 0.0), axis=1,
                       keepdims=True)
        for r in range(T // RB):
            rs = pl.ds(r * RB, RB)
            y = jax.lax.dot_general(hm_ref[rs, :], w2_ref[0],
                                    (((1,), (1,)), ((), ())),
                                    preferred_element_type=_f32)
            y = y * wcol[r * RB:(r + 1) * RB, :]

            @pl.when(e == 0)
            def _():
                out_ref[rs, :] = y

            @pl.when(e > 0)
            def _():
                out_ref[rs, :] += y


def _routed(h2, w1, w2, wgt):
    return pl.pallas_call(
        _routed_body,
        grid=(NW1 + NE,),
        in_specs=[
            pl.BlockSpec((T, D), lambda j: (0, 0)),
            pl.BlockSpec((HH, D), lambda j: (jnp.minimum(j, NW1 - 1), 0)),
            pl.BlockSpec((1, D, HID),
                         lambda j: (jnp.maximum(j - NW1, 0), 0, 0)),
            pl.BlockSpec((T, NE), lambda j: (0, 0)),
        ],
        out_specs=pl.BlockSpec((T, D), lambda j: (0, 0)),
        out_shape=jax.ShapeDtypeStruct((T, D), _f32),
        scratch_shapes=[pltpu.VMEM((T, HID), _f32)],
    )(h2, w1, w2, wgt)


def _final_body(xa_ref, sh_ref, routed_ref, out_ref):
    out_ref[...] = xa_ref[...] + sh_ref[...] + routed_ref[...]


def _final(xa, sh, routed):
    return pl.pallas_call(
        _final_body,
        grid=(NBT,),
        in_specs=[
            pl.BlockSpec((BT, D), lambda i: (i, 0)),
            pl.BlockSpec((BT, D), lambda i: (i, 0)),
            pl.BlockSpec((BT, D), lambda i: (i, 0)),
        ],
        out_specs=pl.BlockSpec((BT, D), lambda i: (i, 0)),
        out_shape=jax.ShapeDtypeStruct((T, D), _f32),
    )(xa, sh, routed)


def kernel(x, Wq, Wk_c, Wv_c, Wk, Wv, Wo, norm1_w, norm2_w, shared_in,
           shared_out, w1_shared, w2_expert, group_gate, expert_gate,
           group_bias, expert_bias):
    x2d = x.reshape(T, D)
    perm = _head_perm()
    wq_p = Wq[perm, :]
    wk_p = Wk[perm, :]
    cos, sin = _rotary_tables()

    q, k, v = _proj(x2d, norm1_w.reshape(1, D), wq_p, Wk_c, Wv_c, wk_p, Wv,
                    cos, sin)
    o = _attn(q, k, v)
    xa, h2, wgt = _post(o, x2d, Wo, norm2_w.reshape(1, D), group_gate,
                        expert_gate, group_bias.reshape(1, NG),
                        expert_bias.reshape(1, NE))
    sh = _ffn_shared(h2, shared_in, shared_out)
    routed = _routed(h2, w1_shared, w2_expert, wgt)
    out = _final(xa, sh, routed)
    return out.reshape(B, T, D)


# speedup vs baseline: 1.1295x; 1.0346x over previous
import functools
import math

import jax
import jax.numpy as jnp
import numpy as np
from jax.experimental import pallas as pl
from jax.experimental.pallas import tpu as pltpu

B, T, D = 1, 2048, 1024
H, DH, DC = 16, 64, 256
HID, NG, EPG, NE, TOPK = 2048, 2, 4, 8, 2
BT = 256
NBT = T // BT

_f32 = jnp.float32


def _dotT(a, b):
    return jax.lax.dot_general(a, b, (((1,), (1,)), ((), ())),
                               preferred_element_type=_f32)


def _rotary_tables():
    inv_freq = 1.0 / (10000.0 ** (np.arange(0, DH, 2, dtype=np.float64) / DH))
    pos = np.arange(T, dtype=np.float64)
    ang = np.einsum('i,j->ij', pos, inv_freq)
    cos = np.cos(ang).astype(np.float32)
    sin = np.sin(ang).astype(np.float32)
    return jnp.asarray(cos), jnp.asarray(sin)


def _head_perm():
    p = np.concatenate([np.arange(0, DH, 2), np.arange(1, DH, 2)])
    full = np.concatenate([h * DH + p for h in range(H)])
    return jnp.asarray(full, dtype=jnp.int32)


def _proj_body(x_ref, n1_ref, wq_ref, wkc_ref, wvc_ref, wk_ref, wv_ref,
               cos_ref, sin_ref, q_ref, k_ref, v_ref):
    x = x_ref[...]
    var = jnp.mean(x * x, axis=-1, keepdims=True)
    h = x / jnp.sqrt(var + 1e-6) * n1_ref[...]
    q = _dotT(h, wq_ref[...])
    kc = _dotT(h, wkc_ref[...])
    vc = _dotT(h, wvc_ref[...])
    k = _dotT(kc, wk_ref[...])
    v = _dotT(vc, wv_ref[...])
    cos = cos_ref[...]
    sin = sin_ref[...]

    scale = 1.0 / math.sqrt(DH)
    for hh in range(H):
        for z, ref, sc in ((q, q_ref, scale), (k, k_ref, 1.0)):
            x1 = z[:, hh * DH:hh * DH + DH // 2]
            x2 = z[:, hh * DH + DH // 2:(hh + 1) * DH]
            ref[hh] = jnp.concatenate(
                [(x1 * cos - x2 * sin) * sc, (x1 * sin + x2 * cos) * sc],
                axis=1)
        v_ref[hh] = v[:, hh * DH:(hh + 1) * DH]


def _proj(x2d, n1, wq_p, wkc, wvc, wk_p, wv, cos, sin):
    grid = (NBT,)
    return pl.pallas_call(
        _proj_body,
        grid=grid,
        in_specs=[
            pl.BlockSpec((BT, D), lambda i: (i, 0)),
            pl.BlockSpec((1, D), lambda i: (0, 0)),
            pl.BlockSpec((D, D), lambda i: (0, 0)),
            pl.BlockSpec((DC, D), lambda i: (0, 0)),
            pl.BlockSpec((DC, D), lambda i: (0, 0)),
            pl.BlockSpec((D, DC), lambda i: (0, 0)),
            pl.BlockSpec((D, DC), lambda i: (0, 0)),
            pl.BlockSpec((BT, DH // 2), lambda i: (i, 0)),
            pl.BlockSpec((BT, DH // 2), lambda i: (i, 0)),
        ],
        out_specs=[
            pl.BlockSpec((H, BT, DH), lambda i: (0, i, 0)),
            pl.BlockSpec((H, BT, DH), lambda i: (0, i, 0)),
            pl.BlockSpec((H, BT, DH), lambda i: (0, i, 0)),
        ],
        out_shape=[jax.ShapeDtypeStruct((H, T, DH), _f32)] * 3,
    )(x2d, n1, wq_p, wkc, wvc, wk_p, wv, cos, sin)


def _attn_body(q_ref, k_ref, v_ref, o_ref):
    q = q_ref[0]
    s = jax.lax.dot_general(q, k_ref[0], (((1,), (1,)), ((), ())),
                            preferred_element_type=_f32)
    m = jnp.max(s, axis=-1, keepdims=True)
    p = jnp.exp(s - m)
    l = jnp.sum(p, axis=-1, keepdims=True)
    o = jax.lax.dot_general(p, v_ref[0], (((1,), (0,)), ((), ())),
                            preferred_element_type=_f32)
    o_ref[0] = o / l


def _attn(q, k, v):
    grid = (H, NBT)
    return pl.pallas_call(
        _attn_body,
        grid=grid,
        in_specs=[
            pl.BlockSpec((1, BT, DH), lambda h, i: (h, i, 0)),
            pl.BlockSpec((1, T, DH), lambda h, i: (h, 0, 0)),
            pl.BlockSpec((1, T, DH), lambda h, i: (h, 0, 0)),
        ],
        out_specs=pl.BlockSpec((1, BT, DH), lambda h, i: (h, i, 0)),
        out_shape=jax.ShapeDtypeStruct((H, T, DH), _f32),
    )(q, k, v)


def _post_body(o_ref, x_ref, wo_ref, n2_ref, gg_ref, eg_ref, gb_ref, eb_ref,
               xa_ref, h2_ref, wgt_ref):
    o2 = jnp.concatenate([o_ref[hh] for hh in range(H)], axis=1)
    xa = x_ref[...] + _dotT(o2, wo_ref[...])
    xa_ref[...] = xa
    var = jnp.mean(xa * xa, axis=-1, keepdims=True)
    h2 = xa / jnp.sqrt(var + 1e-6) * n2_ref[...]
    h2_ref[...] = h2

    glog = _dotT(h2, gg_ref[...]) + gb_ref[...]
    gm = jnp.max(glog, axis=-1, keepdims=True)
    ge = jnp.exp(glog - gm)
    gprobs = ge / jnp.sum(ge, axis=-1, keepdims=True)
    g_idx = (gprobs[:, 1:2] > gprobs[:, 0:1]).astype(jnp.int32)
    g_prob = jnp.max(gprobs, axis=-1, keepdims=True)

    elog = _dotT(h2, eg_ref[...]) + eb_ref[...]
    idx8 = jax.lax.broadcasted_iota(jnp.int32, (BT, NE), 1)
    allowed = (idx8 // EPG) == g_idx
    masked = jnp.where(allowed, elog, -jnp.inf)
    m = jnp.max(masked, axis=-1, keepdims=True)
    ex = jnp.exp(masked - m)
    eprobs = ex / jnp.sum(ex, axis=-1, keepdims=True)
    p = eprobs * g_prob

    m1 = jnp.max(p, axis=-1, keepdims=True)
    i1 = jnp.min(jnp.where(p == m1, idx8, NE), axis=-1, keepdims=True)
    p2 = jnp.where(idx8 == i1, -1.0, p)
    m2 = jnp.max(p2, axis=-1, keepdims=True)
    i2 = jnp.min(jnp.where(p2 == m2, idx8, NE), axis=-1, keepdims=True)
    wgt = jnp.where(idx8 == i1, m1, 0.0) + jnp.where(idx8 == i2, m2, 0.0)
    wgt_ref[...] = wgt


def _post(o, x2d, wo, n2, gg, eg, gb, eb):
    return pl.pallas_call(
        _post_body,
        grid=(NBT,),
        in_specs=[
            pl.BlockSpec((H, BT, DH), lambda i: (0, i, 0)),
            pl.BlockSpec((BT, D), lambda i: (i, 0)),
            pl.BlockSpec((D, D), lambda i: (0, 0)),
            pl.BlockSpec((1, D), lambda i: (0, 0)),
            pl.BlockSpec((NG, D), lambda i: (0, 0)),
            pl.BlockSpec((NE, D), lambda i: (0, 0)),
            pl.BlockSpec((1, NG), lambda i: (0, 0)),
            pl.BlockSpec((1, NE), lambda i: (0, 0)),
        ],
        out_specs=[
            pl.BlockSpec((BT, D), lambda i: (i, 0)),
            pl.BlockSpec((BT, D), lambda i: (i, 0)),
            pl.BlockSpec((BT, NE), lambda i: (i, 0)),
        ],
        out_shape=[
            jax.ShapeDtypeStruct((T, D), _f32),
            jax.ShapeDtypeStruct((T, D), _f32),
            jax.ShapeDtypeStruct((T, NE), _f32),
        ],
    )(o, x2d, wo, n2, gg, eg, gb, eb)


def _swiglu(z):
    a = z[:, :HID]
    b = z[:, HID:]
    return a * jax.lax.logistic(a) * b


def _ffn_final_body(h2_ref, w_ref, so_ref, xa_ref, routed_ref, out_ref):
    hid = _swiglu(_dotT(h2_ref[...], w_ref[...]))
    out_ref[...] = xa_ref[...] + _dotT(hid, so_ref[...]) + routed_ref[...]


def _ffn_final(h2, w, so, xa, routed):
    return pl.pallas_call(
        _ffn_final_body,
        grid=(NBT,),
        in_specs=[
            pl.BlockSpec((BT, D), lambda i: (i, 0)),
            pl.BlockSpec((2 * HID, D), lambda i: (0, 0)),
            pl.BlockSpec((D, HID), lambda i: (0, 0)),
            pl.BlockSpec((BT, D), lambda i: (i, 0)),
            pl.BlockSpec((BT, D), lambda i: (i, 0)),
        ],
        out_specs=pl.BlockSpec((BT, D), lambda i: (i, 0)),
        out_shape=jax.ShapeDtypeStruct((T, D), _f32),
    )(h2, w, so, xa, routed)


HH = HID // 4
NW1 = 8


RB = 512


def _routed_body(h2_ref, w1q_ref, w2_ref, wgt_ref, out_ref, hm_ref):
    j = pl.program_id(0)

    @pl.when(j < NW1 // 2)
    def _():
        sl = pl.ds(j * HH, HH)
        for r in range(T // RB):
            rs = pl.ds(r * RB, RB)
            hm_ref[rs, sl] = _dotT(h2_ref[rs, :], w1q_ref[...])

    @pl.when((j >= NW1 // 2) & (j < NW1))
    def _():
        sl = pl.ds((j - NW1 // 2) * HH, HH)
        for r in range(T // RB):
            rs = pl.ds(r * RB, RB)
            a = hm_ref[rs, sl]
            hm_ref[rs, sl] = a * jax.lax.logistic(a) * _dotT(h2_ref[rs, :],
                                                             w1q_ref[...])

    @pl.when(j >= NW1)
    def _():
        e = j - NW1
        idx8 = jax.lax.broadcasted_iota(jnp.int32, (T, NE), 1)
        wcol = jnp.sum(jnp.where(idx8 == e, wgt_ref[...], 0.0), axis=1,
                       keepdims=True)
        for r in range(T // RB):
            rs = pl.ds(r * RB, RB)
            y = jax.lax.dot_general(hm_ref[rs, :], w2_ref[0],
                                    (((1,), (1,)), ((), ())),
                                    preferred_element_type=_f32)
            y = y * wcol[r * RB:(r + 1) * RB, :]

            @pl.when(e == 0)
            def _():
                out_ref[rs, :] = y

            @pl.when(e > 0)
            def _():
                out_ref[rs, :] += y


def _routed(h2, w1, w2, wgt):
    return pl.pallas_call(
        _routed_body,
        grid=(NW1 + NE,),
        in_specs=[
            pl.BlockSpec((T, D), lambda j: (0, 0)),
            pl.BlockSpec((HH, D), lambda j: (jnp.minimum(j, NW1 - 1), 0)),
            pl.BlockSpec((1, D, HID),
                         lambda j: (jnp.maximum(j - NW1, 0), 0, 0)),
            pl.BlockSpec((T, NE), lambda j: (0, 0)),
        ],
        out_specs=pl.BlockSpec((T, D), lambda j: (0, 0)),
        out_shape=jax.ShapeDtypeStruct((T, D), _f32),
        scratch_shapes=[pltpu.VMEM((T, HID), _f32)],
    )(h2, w1, w2, wgt)


def _final_body(xa_ref, sh_ref, routed_ref, out_ref):
    out_ref[...] = xa_ref[...] + sh_ref[...] + routed_ref[...]


def _final(xa, sh, routed):
    return pl.pallas_call(
        _final_body,
        grid=(NBT,),
        in_specs=[
            pl.BlockSpec((BT, D), lambda i: (i, 0)),
            pl.BlockSpec((BT, D), lambda i: (i, 0)),
            pl.BlockSpec((BT, D), lambda i: (i, 0)),
        ],
        out_specs=pl.BlockSpec((BT, D), lambda i: (i, 0)),
        out_shape=jax.ShapeDtypeStruct((T, D), _f32),
    )(xa, sh, routed)


def kernel(x, Wq, Wk_c, Wv_c, Wk, Wv, Wo, norm1_w, norm2_w, shared_in,
           shared_out, w1_shared, w2_expert, group_gate, expert_gate,
           group_bias, expert_bias):
    x2d = x.reshape(T, D)
    perm = _head_perm()
    wq_p = Wq[perm, :]
    wk_p = Wk[perm, :]
    cos, sin = _rotary_tables()

    q, k, v = _proj(x2d, norm1_w.reshape(1, D), wq_p, Wk_c, Wv_c, wk_p, Wv,
                    cos, sin)
    o = _attn(q, k, v)
    xa, h2, wgt = _post(o, x2d, Wo, norm2_w.reshape(1, D), group_gate,
                        expert_gate, group_bias.reshape(1, NG),
                        expert_bias.reshape(1, NE))
    routed = _routed(h2, w1_shared, w2_expert, wgt)
    out = _ffn_final(h2, shared_in, shared_out, xa, routed)
    return out.reshape(B, T, D)


# speedup vs baseline: 1.1301x; 1.0005x over previous
import functools
import math

import jax
import jax.numpy as jnp
import numpy as np
from jax.experimental import pallas as pl
from jax.experimental.pallas import tpu as pltpu

B, T, D = 1, 2048, 1024
H, DH, DC = 16, 64, 256
HID, NG, EPG, NE, TOPK = 2048, 2, 4, 8, 2
BT = 256
NBT = T // BT

_f32 = jnp.float32


def _dotT(a, b):
    return jax.lax.dot_general(a, b, (((1,), (1,)), ((), ())),
                               preferred_element_type=_f32)


def _rotary_tables():
    inv_freq = 1.0 / (10000.0 ** (np.arange(0, DH, 2, dtype=np.float64) / DH))
    pos = np.arange(T, dtype=np.float64)
    ang = np.einsum('i,j->ij', pos, inv_freq)
    cos = np.cos(ang).astype(np.float32)
    sin = np.sin(ang).astype(np.float32)
    return jnp.asarray(cos), jnp.asarray(sin)


def _head_perm():
    p = np.concatenate([np.arange(0, DH, 2), np.arange(1, DH, 2)])
    full = np.concatenate([h * DH + p for h in range(H)])
    return jnp.asarray(full, dtype=jnp.int32)


def _proj_body(x_ref, n1_ref, wq_ref, wkc_ref, wvc_ref, wk_ref, wv_ref,
               cos_ref, sin_ref, q_ref, k_ref, v_ref):
    x = x_ref[...]
    var = jnp.mean(x * x, axis=-1, keepdims=True)
    h = x / jnp.sqrt(var + 1e-6) * n1_ref[...]
    q = _dotT(h, wq_ref[...])
    kc = _dotT(h, wkc_ref[...])
    vc = _dotT(h, wvc_ref[...])
    k = _dotT(kc, wk_ref[...])
    v = _dotT(vc, wv_ref[...])
    cos = cos_ref[...]
    sin = sin_ref[...]

    scale = 1.0 / math.sqrt(DH)
    for hh in range(H):
        for z, ref, sc in ((q, q_ref, scale), (k, k_ref, 1.0)):
            x1 = z[:, hh * DH:hh * DH + DH // 2]
            x2 = z[:, hh * DH + DH // 2:(hh + 1) * DH]
            ref[hh] = jnp.concatenate(
                [(x1 * cos - x2 * sin) * sc, (x1 * sin + x2 * cos) * sc],
                axis=1)
        v_ref[hh] = v[:, hh * DH:(hh + 1) * DH]


def _proj(x2d, n1, wq_p, wkc, wvc, wk_p, wv, cos, sin):
    grid = (NBT,)
    return pl.pallas_call(
        _proj_body,
        grid=grid,
        in_specs=[
            pl.BlockSpec((BT, D), lambda i: (i, 0)),
            pl.BlockSpec((1, D), lambda i: (0, 0)),
            pl.BlockSpec((D, D), lambda i: (0, 0)),
            pl.BlockSpec((DC, D), lambda i: (0, 0)),
            pl.BlockSpec((DC, D), lambda i: (0, 0)),
            pl.BlockSpec((D, DC), lambda i: (0, 0)),
            pl.BlockSpec((D, DC), lambda i: (0, 0)),
            pl.BlockSpec((BT, DH // 2), lambda i: (i, 0)),
            pl.BlockSpec((BT, DH // 2), lambda i: (i, 0)),
        ],
        out_specs=[
            pl.BlockSpec((H, BT, DH), lambda i: (0, i, 0)),
            pl.BlockSpec((H, BT, DH), lambda i: (0, i, 0)),
            pl.BlockSpec((H, BT, DH), lambda i: (0, i, 0)),
        ],
        out_shape=[jax.ShapeDtypeStruct((H, T, DH), _f32)] * 3,
    )(x2d, n1, wq_p, wkc, wvc, wk_p, wv, cos, sin)


def _attn_body(q_ref, k_ref, v_ref, o_ref):
    q = q_ref[0]
    s = jax.lax.dot_general(q, k_ref[0], (((1,), (1,)), ((), ())),
                            preferred_element_type=_f32)
    m = jnp.max(s, axis=-1, keepdims=True)
    p = jnp.exp(s - m)
    l = jnp.sum(p, axis=-1, keepdims=True)
    o = jax.lax.dot_general(p, v_ref[0], (((1,), (0,)), ((), ())),
                            preferred_element_type=_f32)
    o_ref[0] = o / l


def _attn(q, k, v):
    grid = (H, NBT)
    return pl.pallas_call(
        _attn_body,
        grid=grid,
        in_specs=[
            pl.BlockSpec((1, BT, DH), lambda h, i: (h, i, 0)),
            pl.BlockSpec((1, T, DH), lambda h, i: (h, 0, 0)),
            pl.BlockSpec((1, T, DH), lambda h, i: (h, 0, 0)),
        ],
        out_specs=pl.BlockSpec((1, BT, DH), lambda h, i: (h, i, 0)),
        out_shape=jax.ShapeDtypeStruct((H, T, DH), _f32),
    )(q, k, v)


def _post_body(o_ref, x_ref, wo_ref, n2_ref, gg_ref, eg_ref, gb_ref, eb_ref,
               xa_ref, h2_ref, wgt_ref):
    o2 = jnp.concatenate([o_ref[hh] for hh in range(H)], axis=1)
    xa = x_ref[...] + _dotT(o2, wo_ref[...])
    xa_ref[...] = xa
    var = jnp.mean(xa * xa, axis=-1, keepdims=True)
    h2 = xa / jnp.sqrt(var + 1e-6) * n2_ref[...]
    h2_ref[...] = h2

    glog = _dotT(h2, gg_ref[...]) + gb_ref[...]
    gm = jnp.max(glog, axis=-1, keepdims=True)
    ge = jnp.exp(glog - gm)
    gprobs = ge / jnp.sum(ge, axis=-1, keepdims=True)
    g_idx = (gprobs[:, 1:2] > gprobs[:, 0:1]).astype(jnp.int32)
    g_prob = jnp.max(gprobs, axis=-1, keepdims=True)

    elog = _dotT(h2, eg_ref[...]) + eb_ref[...]
    idx8 = jax.lax.broadcasted_iota(jnp.int32, (BT, NE), 1)
    allowed = (idx8 // EPG) == g_idx
    masked = jnp.where(allowed, elog, -jnp.inf)
    m = jnp.max(masked, axis=-1, keepdims=True)
    ex = jnp.exp(masked - m)
    eprobs = ex / jnp.sum(ex, axis=-1, keepdims=True)
    p = eprobs * g_prob

    m1 = jnp.max(p, axis=-1, keepdims=True)
    i1 = jnp.min(jnp.where(p == m1, idx8, NE), axis=-1, keepdims=True)
    p2 = jnp.where(idx8 == i1, -1.0, p)
    m2 = jnp.max(p2, axis=-1, keepdims=True)
    i2 = jnp.min(jnp.where(p2 == m2, idx8, NE), axis=-1, keepdims=True)
    wgt = jnp.where(idx8 == i1, m1, 0.0) + jnp.where(idx8 == i2, m2, 0.0)
    wgt_ref[...] = wgt


def _post(o, x2d, wo, n2, gg, eg, gb, eb):
    return pl.pallas_call(
        _post_body,
        grid=(NBT,),
        in_specs=[
            pl.BlockSpec((H, BT, DH), lambda i: (0, i, 0)),
            pl.BlockSpec((BT, D), lambda i: (i, 0)),
            pl.BlockSpec((D, D), lambda i: (0, 0)),
            pl.BlockSpec((1, D), lambda i: (0, 0)),
            pl.BlockSpec((NG, D), lambda i: (0, 0)),
            pl.BlockSpec((NE, D), lambda i: (0, 0)),
            pl.BlockSpec((1, NG), lambda i: (0, 0)),
            pl.BlockSpec((1, NE), lambda i: (0, 0)),
        ],
        out_specs=[
            pl.BlockSpec((BT, D), lambda i: (i, 0)),
            pl.BlockSpec((BT, D), lambda i: (i, 0)),
            pl.BlockSpec((BT, NE), lambda i: (i, 0)),
        ],
        out_shape=[
            jax.ShapeDtypeStruct((T, D), _f32),
            jax.ShapeDtypeStruct((T, D), _f32),
            jax.ShapeDtypeStruct((T, NE), _f32),
        ],
    )(o, x2d, wo, n2, gg, eg, gb, eb)


def _swiglu(z):
    a = z[:, :HID]
    b = z[:, HID:]
    return a * jax.lax.logistic(a) * b


def _ffn_final_body(h2_ref, w_ref, so_ref, xa_ref, routed_ref, out_ref):
    hid = _swiglu(_dotT(h2_ref[...], w_ref[...]))
    out_ref[...] = xa_ref[...] + _dotT(hid, so_ref[...]) + routed_ref[...]


def _ffn_final(h2, w, so, xa, routed):
    return pl.pallas_call(
        _ffn_final_body,
        grid=(NBT,),
        in_specs=[
            pl.BlockSpec((BT, D), lambda i: (i, 0)),
            pl.BlockSpec((2 * HID, D), lambda i: (0, 0)),
            pl.BlockSpec((D, HID), lambda i: (0, 0)),
            pl.BlockSpec((BT, D), lambda i: (i, 0)),
            pl.BlockSpec((BT, D), lambda i: (i, 0)),
        ],
        out_specs=pl.BlockSpec((BT, D), lambda i: (i, 0)),
        out_shape=jax.ShapeDtypeStruct((T, D), _f32),
    )(h2, w, so, xa, routed)


HH = HID // 4
NW1 = 8


RB = 512


def _routed_body(h2_ref, w1q_ref, w2_ref, wgt_ref, out_ref, hm_ref):
    j = pl.program_id(0)

    @pl.when(j < NW1 // 2)
    def _():
        sl = pl.ds(j * HH, HH)
        for r in range(T // RB):
            rs = pl.ds(r * RB, RB)
            hm_ref[rs, sl] = _dotT(h2_ref[rs, :], w1q_ref[...])

    @pl.when((j >= NW1 // 2) & (j < NW1))
    def _():
        sl = pl.ds((j - NW1 // 2) * HH, HH)
        for r in range(T // RB):
            rs = pl.ds(r * RB, RB)
            a = hm_ref[rs, sl]
            hm_ref[rs, sl] = a * jax.lax.logistic(a) * _dotT(h2_ref[rs, :],
                                                             w1q_ref[...])

    @pl.when(j >= NW1)
    def _():
        e = j - NW1
        idx8 = jax.lax.broadcasted_iota(jnp.int32, (T, NE), 1)
        wcol = jnp.sum(jnp.where(idx8 == e, wgt_ref[...], 0.0), axis=1,
                       keepdims=True)
        for r in range(T // RB):
            rs = pl.ds(r * RB, RB)
            y = jax.lax.dot_general(hm_ref[rs, :], w2_ref[0],
                                    (((1,), (1,)), ((), ())),
                                    preferred_element_type=_f32)
            y = y * wcol[r * RB:(r + 1) * RB, :]

            @pl.when(e == 0)
            def _():
                out_ref[rs, :] = y

            @pl.when(e > 0)
            def _():
                out_ref[rs, :] += y


def _routed(h2, w1, w2, wgt):
    return pl.pallas_call(
        _routed_body,
        grid=(NW1 + NE,),
        in_specs=[
            pl.BlockSpec((T, D), lambda j: (0, 0)),
            pl.BlockSpec((HH, D), lambda j: (jnp.minimum(j, NW1 - 1), 0)),
            pl.BlockSpec((1, D, HID),
                         lambda j: (jnp.maximum(j - NW1, 0), 0, 0)),
            pl.BlockSpec((T, NE), lambda j: (0, 0)),
        ],
        out_specs=pl.BlockSpec((T, D), lambda j: (0, 0)),
        out_shape=jax.ShapeDtypeStruct((T, D), _f32),
        scratch_shapes=[pltpu.VMEM((T, HID), _f32)],
    )(h2, w1, w2, wgt)


def kernel(x, Wq, Wk_c, Wv_c, Wk, Wv, Wo, norm1_w, norm2_w, shared_in,
           shared_out, w1_shared, w2_expert, group_gate, expert_gate,
           group_bias, expert_bias):
    x2d = x.reshape(T, D)
    perm = _head_perm()
    wq_p = Wq[perm, :]
    wk_p = Wk[perm, :]
    cos, sin = _rotary_tables()

    q, k, v = _proj(x2d, norm1_w.reshape(1, D), wq_p, Wk_c, Wv_c, wk_p, Wv,
                    cos, sin)
    o = _attn(q, k, v)
    xa, h2, wgt = _post(o, x2d, Wo, norm2_w.reshape(1, D), group_gate,
                        expert_gate, group_bias.reshape(1, NG),
                        expert_bias.reshape(1, NE))
    routed = _routed(h2, w1_shared, w2_expert, wgt)
    out = _ffn_final(h2, shared_in, shared_out, xa, routed)
    return out.reshape(B, T, D)
